# Initial kernel scaffold; baseline (speedup 1.0000x reference)
import jax
import jax.numpy as jnp
from jax.experimental import pallas as pl


def kernel(inputs, table):
    raise NotImplementedError("write your pallas kernel here")


# speedup vs baseline: 3.6096x; 3.6096x over previous
import jax
import jax.numpy as jnp
from jax.experimental import pallas as pl
from jax.experimental.pallas import tpu as pltpu
from jax.experimental.pallas import tpu_sc as plsc

_WINDOW = 128


def kernel(inputs, table):
    batch, hist = inputs.shape
    _, embed_dim = table.shape
    num_indices = batch * hist

    lane = 128
    tab128 = jnp.pad(table, ((0, 0), (0, lane - embed_dim)))
    idx_flat = inputs.reshape(1, num_indices)

    mesh = plsc.VectorSubcoreMesh(core_axis_name="core",
                                  subcore_axis_name="subcore")

    @pl.kernel(
        out_type=jax.ShapeDtypeStruct((num_indices, lane), table.dtype),
        mesh=mesh,
    )
    def gather_kernel(table_hbm, idx_hbm, out_hbm):
        def body(idx_vmem, out_vmem):
            pltpu.sync_copy(table_hbm.at[idx_vmem.at[0]], out_vmem)

        pltpu.emit_pipeline(
            body,
            grid=(num_indices // _WINDOW,),
            in_specs=[pl.BlockSpec((1, _WINDOW), index_map=lambda i: (0, i))],
            out_specs=[pl.BlockSpec((_WINDOW, lane),
                                    index_map=lambda i: (i, 0))],
            core_axis_name=("core", "subcore"),
            dimension_semantics=(pltpu.PARALLEL,),
        )(idx_hbm, out_hbm)

    out = gather_kernel(tab128, idx_flat)
    return out[:, :embed_dim].reshape(batch, hist, embed_dim)


# speedup vs baseline: 3.8365x; 1.0629x over previous
import jax
import jax.numpy as jnp
from jax import lax
from jax.experimental import pallas as pl
from jax.experimental.pallas import tpu as pltpu
from jax.experimental.pallas import tpu_sc as plsc

_W = 200
_TILES = 32
_LANE = 128


def kernel(inputs, table):
    batch, hist = inputs.shape
    _, embed_dim = table.shape
    num_indices = batch * hist
    num_windows = num_indices // _W
    per_tile = num_windows // _TILES
    pairs = per_tile // 2

    tab128 = jnp.pad(table, ((0, 0), (0, _LANE - embed_dim)))
    idx_flat = inputs.reshape(num_indices)

    mesh = plsc.VectorSubcoreMesh(core_axis_name="c", subcore_axis_name="s")

    @pl.kernel(
        out_type=jax.ShapeDtypeStruct((num_indices, embed_dim), table.dtype),
        mesh=mesh,
        scratch_types=[
            pltpu.VMEM((_W,), jnp.int32),
            pltpu.VMEM((_W,), jnp.int32),
            pltpu.VMEM((_W, _LANE), jnp.float32),
            pltpu.VMEM((_W, _LANE), jnp.float32),
            pltpu.VMEM((_W, 64), jnp.float32),
            pltpu.VMEM((_W, 64), jnp.float32),
            pltpu.SemaphoreType.DMA,
            pltpu.SemaphoreType.DMA,
            pltpu.SemaphoreType.DMA,
            pltpu.SemaphoreType.DMA,
            pltpu.SemaphoreType.DMA,
            pltpu.SemaphoreType.DMA,
        ],
    )
    def gather_kernel(table_hbm, idx_hbm, out_hbm,
                      idx_a, idx_b, gath_a, gath_b, outb_a, outb_b,
                      si_a, si_b, sg_a, sg_b, so_a, so_b):
        wid = lax.axis_index("c") * 16 + lax.axis_index("s")
        base = wid * per_tile

        def istart(k):
            return (base + k) * _W

        def compact(gath, outb):
            @pl.loop(0, _W)
            def _(r):
                for c in range(embed_dim // 16):
                    outb.at[r, pl.ds(16 * c, 16)][...] = (
                        gath.at[r, pl.ds(16 * c, 16)][...])

        pltpu.async_copy(idx_hbm.at[pl.ds(istart(0), _W)], idx_a, si_a)
        pltpu.async_copy(idx_hbm.at[pl.ds(istart(1), _W)], idx_b, si_b)

        @pl.loop(0, pairs)
        def _(p):
            k_a = 2 * p
            k_b = k_a + 1

            pltpu.make_async_copy(idx_hbm.at[pl.ds(0, _W)], idx_a, si_a).wait()
            cp_a = pltpu.async_copy(table_hbm.at[idx_a], gath_a, sg_a)

            pltpu.make_async_copy(idx_hbm.at[pl.ds(0, _W)], idx_b, si_b).wait()
            cp_b = pltpu.async_copy(table_hbm.at[idx_b], gath_b, sg_b)

            cp_a.wait()

            @pl.when(p > 0)
            def _():
                pltpu.make_async_copy(outb_a,
                                      out_hbm.at[pl.ds(0, _W), :], so_a).wait()

            compact(gath_a, outb_a)
            pltpu.async_copy(outb_a,
                             out_hbm.at[pl.ds(istart(k_a), _W), :], so_a)

            @pl.when(p < pairs - 1)
            def _():
                pltpu.async_copy(idx_hbm.at[pl.ds(istart(k_a + 2), _W)],
                                 idx_a, si_a)

            cp_b.wait()

            @pl.when(p > 0)
            def _():
                pltpu.make_async_copy(outb_b,
                                      out_hbm.at[pl.ds(0, _W), :], so_b).wait()

            compact(gath_b, outb_b)
            pltpu.async_copy(outb_b,
                             out_hbm.at[pl.ds(istart(k_b), _W), :], so_b)

            @pl.when(p < pairs - 1)
            def _():
                pltpu.async_copy(idx_hbm.at[pl.ds(istart(k_b + 2), _W)],
                                 idx_b, si_b)

        pltpu.make_async_copy(outb_a,
                              out_hbm.at[pl.ds(0, _W), :], so_a).wait()
        pltpu.make_async_copy(outb_b,
                              out_hbm.at[pl.ds(0, _W), :], so_b).wait()

    out = gather_kernel(tab128, idx_flat)
    return out.reshape(batch, hist, embed_dim)


# speedup vs baseline: 5.0451x; 1.3150x over previous
import jax
import jax.numpy as jnp
from jax import lax
from jax.experimental import pallas as pl
from jax.experimental.pallas import tpu as pltpu
from jax.experimental.pallas import tpu_sc as plsc

_W = 200
_TILES = 32
_LANE = 128


def kernel(inputs, table):
    batch, hist = inputs.shape
    _, embed_dim = table.shape
    num_indices = batch * hist
    num_windows = num_indices // _W
    per_tile = num_windows // _TILES
    pairs = per_tile // 2
    bpw = _W // hist
    nchunk = embed_dim // 16

    tab128 = jnp.pad(table, ((0, 0), (0, _LANE - embed_dim)))
    idx_flat = inputs.reshape(num_indices)

    mesh = plsc.VectorSubcoreMesh(core_axis_name="c", subcore_axis_name="s")

    @pl.kernel(
        out_type=jax.ShapeDtypeStruct((batch, hist, embed_dim), table.dtype),
        mesh=mesh,
        scratch_types=[
            pltpu.VMEM((_W,), jnp.int32),
            pltpu.VMEM((_W,), jnp.int32),
            pltpu.VMEM((_W, _LANE), jnp.float32),
            pltpu.VMEM((_W, _LANE), jnp.float32),
            pltpu.VMEM((bpw, hist, 64), jnp.float32),
            pltpu.VMEM((bpw, hist, 64), jnp.float32),
            pltpu.SemaphoreType.DMA,
            pltpu.SemaphoreType.DMA,
            pltpu.SemaphoreType.DMA,
            pltpu.SemaphoreType.DMA,
            pltpu.SemaphoreType.DMA,
            pltpu.SemaphoreType.DMA,
        ],
    )
    def gather_kernel(table_hbm, idx_hbm, out_hbm,
                      idx_a, idx_b, gath_a, gath_b, outb_a, outb_b,
                      si_a, si_b, sg_a, sg_b, so_a, so_b):
        wid = lax.axis_index("c") * 16 + lax.axis_index("s")
        base = wid * per_tile

        def istart(k):
            return (base + k) * _W

        def bstart(k):
            return (base + k) * bpw

        def compact(gath, outb):
            @pl.loop(0, bpw)
            def _(b):
                @pl.loop(0, hist)
                def _(r):
                    for c in range(nchunk):
                        outb.at[b, r, pl.ds(16 * c, 16)][...] = (
                            gath.at[b * hist + r, pl.ds(16 * c, 16)][...])

        pltpu.async_copy(idx_hbm.at[pl.ds(istart(0), _W)], idx_a, si_a)
        pltpu.async_copy(idx_hbm.at[pl.ds(istart(1), _W)], idx_b, si_b)

        @pl.loop(0, pairs)
        def _(p):
            k_a = 2 * p
            k_b = k_a + 1

            pltpu.make_async_copy(idx_hbm.at[pl.ds(0, _W)], idx_a, si_a).wait()
            cp_a = pltpu.async_copy(table_hbm.at[idx_a], gath_a, sg_a)

            pltpu.make_async_copy(idx_hbm.at[pl.ds(0, _W)], idx_b, si_b).wait()
            cp_b = pltpu.async_copy(table_hbm.at[idx_b], gath_b, sg_b)

            cp_a.wait()

            @pl.when(p > 0)
            def _():
                pltpu.make_async_copy(outb_a, out_hbm.at[pl.ds(0, bpw), :, :],
                                      so_a).wait()

            compact(gath_a, outb_a)
            pltpu.async_copy(outb_a, out_hbm.at[pl.ds(bstart(k_a), bpw), :, :],
                             so_a)

            @pl.when(p < pairs - 1)
            def _():
                pltpu.async_copy(idx_hbm.at[pl.ds(istart(k_a + 2), _W)],
                                 idx_a, si_a)

            cp_b.wait()

            @pl.when(p > 0)
            def _():
                pltpu.make_async_copy(outb_b, out_hbm.at[pl.ds(0, bpw), :, :],
                                      so_b).wait()

            compact(gath_b, outb_b)
            pltpu.async_copy(outb_b, out_hbm.at[pl.ds(bstart(k_b), bpw), :, :],
                             so_b)

            @pl.when(p < pairs - 1)
            def _():
                pltpu.async_copy(idx_hbm.at[pl.ds(istart(k_b + 2), _W)],
                                 idx_b, si_b)

        pltpu.make_async_copy(outb_a, out_hbm.at[pl.ds(0, bpw), :, :],
                              so_a).wait()
        pltpu.make_async_copy(outb_b, out_hbm.at[pl.ds(0, bpw), :, :],
                              so_b).wait()

    return gather_kernel(tab128, idx_flat)
